# baseline (speedup 1.0000x reference)
import functools

import jax
import jax.numpy as jnp
from jax import lax
from jax.experimental import pallas as pl
from jax.experimental.pallas import tpu as pltpu
from jax.experimental.pallas import tpu_sc as plsc

N = 1_600_000
B = 50_000
NC, NS, L = 2, 16, 16
NW = NC * NS
CHUNK = N // NW
T = 2_000
NT = CHUNK // T
NRING = 5
WBIG = 2_048
BP = 52_224
SB = BP // NS

_mesh = plsc.VectorSubcoreMesh(core_axis_name="c", subcore_axis_name="s")


def _zero_fill(buf, n):
    zeros = jnp.zeros((L,), jnp.float32)

    @plsc.parallel_loop(0, n // L, unroll=4)
    def _(i):
        buf[pl.ds(i * L, L)] = zeros


def _fused_body(chi_hbm, hard_hbm, ids_hbm, tc_hbm, parts_hbm, out_hbm,
                idx_v, a_v, b_v, o_v, hv_v, cv_v, z_v, wh_v, wc_v, widx_v,
                g_v, rwin_v, pa_v, pb_v, tc_v, acc_h, acc_c, lsems, stsems,
                xsem):
    cid = lax.axis_index("c")
    sid = lax.axis_index("s")
    wid = sid * NC + cid

    def load_descs(t, b):
        sl = pl.ds(wid * CHUNK + t * T, T)
        return (pltpu.make_async_copy(ids_hbm.at[sl], idx_v[b], lsems[b]),
                pltpu.make_async_copy(chi_hbm.at[sl], a_v[b], lsems[b]),
                pltpu.make_async_copy(hard_hbm.at[sl], b_v[b], lsems[b]))

    def issue_loads(t, b):
        for d in load_descs(t, b):
            d.start()

    def wait_loads(t, b):
        for d in load_descs(t, b):
            d.wait()

    def store_desc(t, b):
        return pltpu.make_async_copy(
            o_v[b], out_hbm.at[pl.ds(wid * CHUNK + t * T, T)], stsems[b])

    _zero_fill(z_v, SB)
    pltpu.sync_copy(z_v, acc_h.at[pl.ds(sid * SB, SB)])
    pltpu.sync_copy(z_v, acc_c.at[pl.ds(sid * SB, SB)])
    _zero_fill(wh_v, WBIG)
    _zero_fill(wc_v, WBIG)
    plsc.subcore_barrier()

    pltpu.sync_copy(ids_hbm.at[pl.ds(wid * CHUNK, L)], widx_v.at[pl.ds(0, L)])
    pltpu.sync_copy(ids_hbm.at[pl.ds(wid * CHUNK + CHUNK - L, L)],
                    widx_v.at[pl.ds(L, L)])
    lo = widx_v[pl.ds(0, L)][0]
    hi = widx_v[pl.ds(L, L)][L - 1]
    lo8 = pl.multiple_of((lo // 8) * 8, 8)
    fits = hi - lo8 < WBIG

    iota = lax.broadcasted_iota(jnp.int32, (L,), 0)
    nxt = jnp.minimum(iota + 1, L - 1)
    prv = jnp.maximum(iota - 1, 0)

    @pl.when(fits)
    def _():
        issue_loads(0, 0)
        issue_loads(1, 1)

        @pl.loop(0, NT, step=NRING)
        def _(t0):
            for b in range(NRING):
                t = t0 + b

                @pl.when(t + 2 < NT)
                def _():
                    issue_loads(t + 2, (b + 2) % NRING)

                wait_loads(t, b)
                ir, ar, br = idx_v[b], a_v[b], b_v[b]

                @plsc.parallel_loop(0, T // L, unroll=5)
                def _(i):
                    sl = pl.ds(i * L, L)
                    sv = ir[sl]
                    li = sv - lo
                    hinv = 1.0 / br[sl]
                    ch = plsc.cumsum(hinv)
                    cc = plsc.cumsum(ar[sl] * hinv)
                    end = (sv != jnp.take_along_axis(sv, nxt, axis=0)) | (
                        iota == L - 1)
                    start = (sv != jnp.take_along_axis(sv, prv, axis=0))
                    chp = jnp.take_along_axis(ch, prv, axis=0)
                    ccp = jnp.take_along_axis(cc, prv, axis=0)
                    plsc.addupdate_scatter(wh_v, [li], ch, mask=end)
                    plsc.addupdate_scatter(wc_v, [li], cc, mask=end)
                    plsc.addupdate_scatter(wh_v, [li], -chp, mask=start)
                    plsc.addupdate_scatter(wc_v, [li], -ccp, mask=start)

        @plsc.parallel_loop(0, WBIG // L, unroll=4)
        def _(i):
            widx_v[pl.ds(i * L, L)] = lo + i * L + iota

        pltpu.sync_copy(wh_v, acc_h.at[widx_v], add=True)
        pltpu.sync_copy(wc_v, acc_c.at[widx_v], add=True)

    @pl.when(jnp.logical_not(fits))
    def _():
        @pl.loop(0, NT)
        def _(t):
            sl = pl.ds(wid * CHUNK + t * T, T)
            pltpu.sync_copy(ids_hbm.at[sl], idx_v[0])
            pltpu.sync_copy(chi_hbm.at[sl], a_v[0])
            pltpu.sync_copy(hard_hbm.at[sl], b_v[0])

            @plsc.parallel_loop(0, T // L, unroll=5)
            def _(i):
                s2 = pl.ds(i * L, L)
                hinv = 1.0 / b_v[0][s2]
                hv_v[s2] = hinv
                cv_v[s2] = a_v[0][s2] * hinv

            pltpu.sync_copy(hv_v.at[pl.ds(0, T)], acc_h.at[idx_v[0]],
                            add=True)
            pltpu.sync_copy(cv_v.at[pl.ds(0, T)], acc_c.at[idx_v[0]],
                            add=True)

    plsc.subcore_barrier()
    pltpu.sync_copy(acc_h.at[pl.ds(sid * SB, SB)], hv_v)
    pltpu.sync_copy(hv_v, parts_hbm.at[pl.ds(2 * cid * BP + sid * SB, SB)])
    pltpu.sync_copy(acc_c.at[pl.ds(sid * SB, SB)], cv_v)
    pltpu.sync_copy(cv_v,
                    parts_hbm.at[pl.ds((2 * cid + 1) * BP + sid * SB, SB)])

    @pl.when(fits)
    def _():
        issue_loads(0, 0)
        issue_loads(1, 1)

    plsc.subcore_barrier()
    pltpu.semaphore_signal(xsem, 1, core_index=1 - cid)
    pltpu.semaphore_wait(xsem, 1)

    sl = pl.ds(sid * SB, SB)
    pltpu.sync_copy(parts_hbm.at[pl.ds(0 * BP + sid * SB, SB)], pa_v)
    pltpu.sync_copy(parts_hbm.at[pl.ds(2 * BP + sid * SB, SB)], pb_v)
    pltpu.sync_copy(parts_hbm.at[pl.ds(1 * BP + sid * SB, SB)], z_v)
    pltpu.sync_copy(parts_hbm.at[pl.ds(3 * BP + sid * SB, SB)], g_v)
    pltpu.sync_copy(tc_hbm, tc_v)
    tc = tc_v[...]

    @plsc.parallel_loop(0, SB // L, unroll=4)
    def _(i):
        s2 = pl.ds(i * L, L)
        sh = pa_v[s2] + pb_v[s2]
        sc = z_v[s2] + g_v[s2]
        pa_v[s2] = (tc + sc) / sh

    pltpu.sync_copy(pa_v, acc_h.at[sl])
    plsc.subcore_barrier()

    @pl.when(fits)
    def _():
        pltpu.sync_copy(acc_h.at[pl.ds(lo8, WBIG)], rwin_v)

        @pl.loop(0, NT, step=NRING)
        def _(t0):
            for b in range(NRING):
                t = t0 + b

                @pl.when(t >= 2)
                def _():
                    store_desc(t - 2, (b - 2) % NRING).wait()

                @pl.when(t + 2 < NT)
                def _():
                    issue_loads(t + 2, (b + 2) % NRING)

                wait_loads(t, b)
                ir, ar, br, orr = idx_v[b], a_v[b], b_v[b], o_v[b]

                @plsc.parallel_loop(0, T // L, unroll=5)
                def _(i):
                    s2 = pl.ds(i * L, L)
                    li = ir[s2] - lo8
                    r = plsc.load_gather(rwin_v, [li])
                    orr[s2] = (r - ar[s2]) / br[s2]

                store_desc(t, b).start()

        store_desc(NT - 2, (NT - 2) % NRING).wait()
        store_desc(NT - 1, (NT - 1) % NRING).wait()

    @pl.when(jnp.logical_not(fits))
    def _():
        @pl.loop(0, NT)
        def _(t):
            sl2 = pl.ds(wid * CHUNK + t * T, T)
            pltpu.sync_copy(ids_hbm.at[sl2], idx_v[0])
            pltpu.sync_copy(chi_hbm.at[sl2], a_v[0])
            pltpu.sync_copy(hard_hbm.at[sl2], b_v[0])
            pltpu.sync_copy(acc_h.at[idx_v[0]], g_v.at[pl.ds(0, T)])

            @plsc.parallel_loop(0, T // L, unroll=5)
            def _(i):
                s2 = pl.ds(i * L, L)
                o_v[0][s2] = (g_v[s2] - a_v[0][s2]) / b_v[0][s2]

            pltpu.sync_copy(o_v[0], out_hbm.at[sl2])


_fused = functools.partial(
    pl.kernel,
    out_type=(jax.ShapeDtypeStruct((NC * 2 * BP,), jnp.float32),
              jax.ShapeDtypeStruct((N,), jnp.float32)),
    mesh=_mesh,
    compiler_params=pltpu.CompilerParams(needs_layout_passes=False),
    scratch_types=[
        [pltpu.VMEM((T,), jnp.int32)] * NRING,
        [pltpu.VMEM((T,), jnp.float32)] * NRING,
        [pltpu.VMEM((T,), jnp.float32)] * NRING,
        [pltpu.VMEM((T,), jnp.float32)] * NRING,
        pltpu.VMEM((SB,), jnp.float32),
        pltpu.VMEM((SB,), jnp.float32),
        pltpu.VMEM((SB,), jnp.float32),
        pltpu.VMEM((WBIG,), jnp.float32),
        pltpu.VMEM((WBIG,), jnp.float32),
        pltpu.VMEM((WBIG,), jnp.int32),
        pltpu.VMEM((SB,), jnp.float32),
        pltpu.VMEM((WBIG,), jnp.float32),
        pltpu.VMEM((SB,), jnp.float32),
        pltpu.VMEM((SB,), jnp.float32),
        pltpu.VMEM((L,), jnp.float32),
        pltpu.VMEM_SHARED((BP,), jnp.float32),
        pltpu.VMEM_SHARED((BP,), jnp.float32),
        [pltpu.SemaphoreType.DMA] * NRING,
        [pltpu.SemaphoreType.DMA] * NRING,
        pltpu.SemaphoreType.REGULAR,
    ],
)(_fused_body)


def kernel(chi, hardness, segment_ids, total_charge):
    tc_vec = jnp.broadcast_to(total_charge, (L,)).astype(jnp.float32)
    _, charge = _fused(chi, hardness, segment_ids, tc_vec)
    return charge


# speedup vs baseline: 1.0236x; 1.0236x over previous
import functools

import jax
import jax.numpy as jnp
from jax import lax
from jax.experimental import pallas as pl
from jax.experimental.pallas import tpu as pltpu
from jax.experimental.pallas import tpu_sc as plsc

N = 1_600_000
B = 50_000
NC, NS, L = 2, 16, 16
NW = NC * NS
CHUNK = N // NW
T = 2_000
NT = CHUNK // T
NRING = 5
WBIG = 2_048
BP = 52_224
SB = BP // NS

_mesh = plsc.VectorSubcoreMesh(core_axis_name="c", subcore_axis_name="s")


def _zero_fill(buf, n):
    zeros = jnp.zeros((L,), jnp.float32)

    @plsc.parallel_loop(0, n // L, unroll=4)
    def _(i):
        buf[pl.ds(i * L, L)] = zeros


def _fused_body(chi_hbm, hard_hbm, ids_hbm, tc_hbm, parts_hbm, out_hbm,
                idx_v, a_v, b_v, o_v, hv_v, cv_v, z_v, wh_v, wc_v, widx_v,
                g_v, rwin_v, pa_v, pb_v, tc_v, acc_h, acc_c, lsems, stsems,
                xsem):
    cid = lax.axis_index("c")
    sid = lax.axis_index("s")
    wid = sid * NC + cid

    def load_descs(t, b):
        sl = pl.ds(wid * CHUNK + t * T, T)
        return (pltpu.make_async_copy(ids_hbm.at[sl], idx_v[b], lsems[b]),
                pltpu.make_async_copy(chi_hbm.at[sl], a_v[b], lsems[b]),
                pltpu.make_async_copy(hard_hbm.at[sl], b_v[b], lsems[b]))

    def issue_loads(t, b):
        for d in load_descs(t, b):
            d.start()

    def wait_loads(t, b):
        for d in load_descs(t, b):
            d.wait()

    def store_desc(t, b):
        return pltpu.make_async_copy(
            o_v[b], out_hbm.at[pl.ds(wid * CHUNK + t * T, T)], stsems[b])

    _zero_fill(z_v, SB)
    pltpu.sync_copy(z_v, acc_h.at[pl.ds(sid * SB, SB)])
    pltpu.sync_copy(z_v, acc_c.at[pl.ds(sid * SB, SB)])
    _zero_fill(wh_v, WBIG)
    _zero_fill(wc_v, WBIG)
    plsc.subcore_barrier()

    pltpu.sync_copy(ids_hbm.at[pl.ds(wid * CHUNK, L)], widx_v.at[pl.ds(0, L)])
    pltpu.sync_copy(ids_hbm.at[pl.ds(wid * CHUNK + CHUNK - L, L)],
                    widx_v.at[pl.ds(L, L)])
    lo = widx_v[pl.ds(0, L)][0]
    hi = widx_v[pl.ds(L, L)][L - 1]
    lo8 = pl.multiple_of((lo // 8) * 8, 8)
    fits = hi - lo8 < WBIG

    iota = lax.broadcasted_iota(jnp.int32, (L,), 0)
    nxt = jnp.minimum(iota + 1, L - 1)
    prv = jnp.maximum(iota - 1, 0)

    @pl.when(fits)
    def _():
        issue_loads(0, 0)
        issue_loads(1, 1)

        @pl.loop(0, NT, step=NRING)
        def _(t0):
            for b in range(NRING):
                t = t0 + b

                @pl.when(t + 2 < NT)
                def _():
                    issue_loads(t + 2, (b + 2) % NRING)

                wait_loads(t, b)
                ir, ar, br = idx_v[b], a_v[b], b_v[b]

                @plsc.parallel_loop(0, T // L, unroll=5)
                def _(i):
                    sl = pl.ds(i * L, L)
                    sv = ir[sl]
                    li = sv - lo
                    hinv = 1.0 / br[sl]
                    ch = plsc.cumsum(hinv)
                    cc = plsc.cumsum(ar[sl] * hinv)
                    end = (sv != jnp.take_along_axis(sv, nxt, axis=0)) | (
                        iota == L - 1)
                    m = jnp.where(end, iota, -1)
                    p = jnp.take_along_axis(plsc.cummax(m), prv, axis=0)
                    p = jnp.where(iota == 0, -1, p)
                    has_p = p >= 0
                    pc = jnp.maximum(p, 0)
                    bh = jnp.where(has_p,
                                   jnp.take_along_axis(ch, pc, axis=0), 0.0)
                    bc = jnp.where(has_p,
                                   jnp.take_along_axis(cc, pc, axis=0), 0.0)
                    plsc.addupdate_scatter(wh_v, [lic := li], ch - bh,
                                           mask=end)
                    plsc.addupdate_scatter(wc_v, [lic], cc - bc, mask=end)

        @plsc.parallel_loop(0, WBIG // L, unroll=4)
        def _(i):
            widx_v[pl.ds(i * L, L)] = lo + i * L + iota

        dh = pltpu.async_copy(wh_v, acc_h.at[widx_v], stsems[0], add=True)
        dc = pltpu.async_copy(wc_v, acc_c.at[widx_v], stsems[1], add=True)
        dh.wait()
        dc.wait()

    @pl.when(jnp.logical_not(fits))
    def _():
        @pl.loop(0, NT)
        def _(t):
            sl = pl.ds(wid * CHUNK + t * T, T)
            pltpu.sync_copy(ids_hbm.at[sl], idx_v[0])
            pltpu.sync_copy(chi_hbm.at[sl], a_v[0])
            pltpu.sync_copy(hard_hbm.at[sl], b_v[0])

            @plsc.parallel_loop(0, T // L, unroll=5)
            def _(i):
                s2 = pl.ds(i * L, L)
                hinv = 1.0 / b_v[0][s2]
                hv_v[s2] = hinv
                cv_v[s2] = a_v[0][s2] * hinv

            pltpu.sync_copy(hv_v.at[pl.ds(0, T)], acc_h.at[idx_v[0]],
                            add=True)
            pltpu.sync_copy(cv_v.at[pl.ds(0, T)], acc_c.at[idx_v[0]],
                            add=True)

    plsc.subcore_barrier()
    pltpu.sync_copy(acc_h.at[pl.ds(sid * SB, SB)], hv_v)
    d1 = pltpu.async_copy(
        hv_v, parts_hbm.at[pl.ds(2 * cid * BP + sid * SB, SB)], stsems[0])
    pltpu.sync_copy(acc_c.at[pl.ds(sid * SB, SB)], cv_v)
    d2 = pltpu.async_copy(
        cv_v, parts_hbm.at[pl.ds((2 * cid + 1) * BP + sid * SB, SB)],
        stsems[1])
    pltpu.sync_copy(acc_h.at[pl.ds(sid * SB, SB)], pa_v)
    pltpu.sync_copy(acc_c.at[pl.ds(sid * SB, SB)], z_v)
    pltpu.sync_copy(tc_hbm, tc_v)
    d1.wait()
    d2.wait()

    @pl.when(fits)
    def _():
        issue_loads(0, 0)
        issue_loads(1, 1)

    plsc.subcore_barrier()
    pltpu.semaphore_signal(xsem, 1, core_index=1 - cid)
    pltpu.semaphore_wait(xsem, 1)

    oc = 1 - cid
    sl = pl.ds(sid * SB, SB)
    pltpu.sync_copy(parts_hbm.at[pl.ds(2 * oc * BP + sid * SB, SB)], pb_v)
    pltpu.sync_copy(parts_hbm.at[pl.ds((2 * oc + 1) * BP + sid * SB, SB)],
                    g_v)
    tc = tc_v[...]

    @plsc.parallel_loop(0, SB // L, unroll=4)
    def _(i):
        s2 = pl.ds(i * L, L)
        sh = pa_v[s2] + pb_v[s2]
        sc = z_v[s2] + g_v[s2]
        pa_v[s2] = (tc + sc) / sh

    pltpu.sync_copy(pa_v, acc_h.at[sl])
    plsc.subcore_barrier()

    @pl.when(fits)
    def _():
        pltpu.sync_copy(acc_h.at[pl.ds(lo8, WBIG)], rwin_v)

        @pl.loop(0, NT, step=NRING)
        def _(t0):
            for b in range(NRING):
                t = t0 + b

                @pl.when(t >= 2)
                def _():
                    store_desc(t - 2, (b - 2) % NRING).wait()

                @pl.when(t + 2 < NT)
                def _():
                    issue_loads(t + 2, (b + 2) % NRING)

                wait_loads(t, b)
                ir, ar, br, orr = idx_v[b], a_v[b], b_v[b], o_v[b]

                @plsc.parallel_loop(0, T // L, unroll=5)
                def _(i):
                    s2 = pl.ds(i * L, L)
                    li = ir[s2] - lo8
                    r = plsc.load_gather(rwin_v, [li])
                    orr[s2] = (r - ar[s2]) / br[s2]

                store_desc(t, b).start()

        store_desc(NT - 2, (NT - 2) % NRING).wait()
        store_desc(NT - 1, (NT - 1) % NRING).wait()

    @pl.when(jnp.logical_not(fits))
    def _():
        @pl.loop(0, NT)
        def _(t):
            sl2 = pl.ds(wid * CHUNK + t * T, T)
            pltpu.sync_copy(ids_hbm.at[sl2], idx_v[0])
            pltpu.sync_copy(chi_hbm.at[sl2], a_v[0])
            pltpu.sync_copy(hard_hbm.at[sl2], b_v[0])
            pltpu.sync_copy(acc_h.at[idx_v[0]], g_v.at[pl.ds(0, T)])

            @plsc.parallel_loop(0, T // L, unroll=5)
            def _(i):
                s2 = pl.ds(i * L, L)
                o_v[0][s2] = (g_v[s2] - a_v[0][s2]) / b_v[0][s2]

            pltpu.sync_copy(o_v[0], out_hbm.at[sl2])


_fused = functools.partial(
    pl.kernel,
    out_type=(jax.ShapeDtypeStruct((NC * 2 * BP,), jnp.float32),
              jax.ShapeDtypeStruct((N,), jnp.float32)),
    mesh=_mesh,
    compiler_params=pltpu.CompilerParams(needs_layout_passes=False),
    scratch_types=[
        [pltpu.VMEM((T,), jnp.int32)] * NRING,
        [pltpu.VMEM((T,), jnp.float32)] * NRING,
        [pltpu.VMEM((T,), jnp.float32)] * NRING,
        [pltpu.VMEM((T,), jnp.float32)] * NRING,
        pltpu.VMEM((SB,), jnp.float32),
        pltpu.VMEM((SB,), jnp.float32),
        pltpu.VMEM((SB,), jnp.float32),
        pltpu.VMEM((WBIG,), jnp.float32),
        pltpu.VMEM((WBIG,), jnp.float32),
        pltpu.VMEM((WBIG,), jnp.int32),
        pltpu.VMEM((SB,), jnp.float32),
        pltpu.VMEM((WBIG,), jnp.float32),
        pltpu.VMEM((SB,), jnp.float32),
        pltpu.VMEM((SB,), jnp.float32),
        pltpu.VMEM((L,), jnp.float32),
        pltpu.VMEM_SHARED((BP,), jnp.float32),
        pltpu.VMEM_SHARED((BP,), jnp.float32),
        [pltpu.SemaphoreType.DMA] * NRING,
        [pltpu.SemaphoreType.DMA] * NRING,
        pltpu.SemaphoreType.REGULAR,
    ],
)(_fused_body)


def kernel(chi, hardness, segment_ids, total_charge):
    tc_vec = jnp.broadcast_to(total_charge, (L,)).astype(jnp.float32)
    _, charge = _fused(chi, hardness, segment_ids, tc_vec)
    return charge
